# baseline (speedup 1.0000x reference)
import dataclasses
import functools

import jax
import jax.numpy as jnp
from jax import lax
from jax.experimental import pallas as pl
from jax.experimental.pallas import tpu as pltpu
from jax.experimental.pallas import tpu_sc as plsc

_N_SUBS = 64
_D_MID = 64
_D_OUT = 64
_ST_W = 256
_NW = 32
_RPW = _N_SUBS // _NW


def _pass1_body(x_ref, sub_ref, st_ref):
    x = x_ref[...]
    b, d_in = x.shape
    xh = x.astype(jnp.bfloat16)
    sub = sub_ref[0, 0, :]
    segT = jax.lax.broadcasted_iota(jnp.int32, (_N_SUBS, b), 0)
    maskT = sub[None, :] == segT
    mseg = jax.lax.dot_general(
        maskT.astype(jnp.bfloat16), xh, (((1,), (0,)), ((), ())),
        preferred_element_type=jnp.float32)
    cnt = jnp.sum(maskT.astype(jnp.float32), axis=1, keepdims=True)
    r0 = jnp.sum(x[0:1, :], axis=1, keepdims=True)

    st_ref[0, :, :d_in] = mseg
    st_ref[0, :, d_in:d_in + 1] = cnt
    st_ref[0, :, d_in + 1:d_in + 2] = jnp.broadcast_to(r0, (_N_SUBS, 1))
    st_ref[0, :, d_in + 2:] = jnp.zeros(
        (_N_SUBS, _ST_W - d_in - 2), jnp.float32)


def _sc_mid_body(st_hbm, g_hbm, t2_hbm, part_v, g_v, t2_v, sem):
    wid = lax.axis_index("s") * 2 + lax.axis_index("c")
    row0 = wid * _RPW
    nb = st_hbm.shape[0]
    d_in = 128
    pltpu.async_copy(
        st_hbm.at[:, pl.ds(row0, _RPW), :], part_v, sem).wait()
    pltpu.async_copy(g_hbm, g_v, sem).wait()
    gvec = g_v[...]

    for row in range(_RPW):
        tot_s = jnp.zeros((16,), jnp.float32)
        tot_aux = jnp.zeros((16,), jnp.float32)
        for p in range(nb):
            for j in range(8):
                tot_s = tot_s + part_v[p, row, pl.ds(16 * j, 16)]
            tot_aux = tot_aux + part_v[p, row, pl.ds(d_in, 16)]
        S = jnp.sum(tot_s)
        lane = lax.iota(jnp.int32, 16)
        C = jnp.sum(jnp.where(lane == 0, tot_aux, 0.0))
        r0 = jnp.sum(jnp.where(lane == 1, tot_aux, 0.0))
        Sv = jnp.full((16,), S, jnp.float32)
        Cv = jnp.full((16,), C, jnp.float32)
        r0v = jnp.full((16,), r0, jnp.float32)
        mv = jnp.where(Cv > 0.0, Sv / jnp.maximum(Cv, 1.0), r0v)
        tvec = jnp.maximum(gvec * mv, 0.0) * float(_D_MID)
        for j in range(4):
            t2_v[row, pl.ds(16 * j, 16)] = tvec

    pltpu.async_copy(t2_v, t2_hbm.at[pl.ds(row0, _RPW), :], sem).wait()


def _pass2_body(l_ref, x_ref, sub_ref, t2_ref, out_ref):
    x = x_ref[...]
    b, d_in = x.shape
    xh = x.astype(jnp.bfloat16)
    xl = (x - xh.astype(jnp.float32)).astype(jnp.bfloat16)
    ones = jnp.ones((d_in, _D_OUT), jnp.bfloat16)
    dot = lambda a, c: jax.lax.dot_general(
        a, c, (((1,), (0,)), ((), ())), preferred_element_type=jnp.float32)
    rB = dot(xh, ones) + dot(xl, ones)

    t2 = t2_ref[...].astype(jnp.bfloat16)
    sub = sub_ref[0, 0, :]
    seg = jax.lax.broadcasted_iota(jnp.int32, (b, _N_SUBS), 1)
    mask = (sub[:, None] == seg).astype(jnp.bfloat16)
    gB = dot(mask, t2)
    lam = l_ref[0, 0]
    out_ref[...] = jnp.maximum(lam * (rB + gB), 0.0)


def kernel(x, sub, Gamma, Lambda):
    n, d_in = x.shape
    B = 16000
    nb = n // B
    sub3 = sub.reshape(nb, 1, B)
    g16 = jnp.broadcast_to(Gamma.reshape(1), (16,))
    lv = jnp.broadcast_to(Lambda.reshape(1, 1), (8, 128))
    par = pltpu.CompilerParams(dimension_semantics=("parallel",))

    st = pl.pallas_call(
        _pass1_body,
        grid=(nb,),
        in_specs=[
            pl.BlockSpec((B, d_in), lambda i: (i, 0)),
            pl.BlockSpec((1, 1, B), lambda i: (i, 0, 0)),
        ],
        out_specs=pl.BlockSpec((1, _N_SUBS, _ST_W), lambda i: (i, 0, 0)),
        out_shape=jax.ShapeDtypeStruct((nb, _N_SUBS, _ST_W), jnp.float32),
        compiler_params=par,
    )(x, sub3)

    mesh = plsc.VectorSubcoreMesh(core_axis_name="c", subcore_axis_name="s")
    sc_params = pltpu.CompilerParams()
    if "needs_layout_passes" in pltpu.CompilerParams.__dataclass_fields__:
        sc_params = dataclasses.replace(sc_params, needs_layout_passes=False)
    sc_mid = functools.partial(
        pl.kernel,
        out_type=jax.ShapeDtypeStruct((_N_SUBS, _N_SUBS), jnp.float32),
        mesh=mesh,
        compiler_params=sc_params,
        scratch_types=[
            pltpu.VMEM((nb, _RPW, _ST_W), jnp.float32),
            pltpu.VMEM((16,), jnp.float32),
            pltpu.VMEM((_RPW, _N_SUBS), jnp.float32),
            pltpu.SemaphoreType.DMA,
        ],
    )(_sc_mid_body)
    t2 = sc_mid(st, g16)

    out = pl.pallas_call(
        _pass2_body,
        grid=(nb,),
        in_specs=[
            pl.BlockSpec((8, 128), lambda i: (0, 0)),
            pl.BlockSpec((B, d_in), lambda i: (i, 0)),
            pl.BlockSpec((1, 1, B), lambda i: (i, 0, 0)),
            pl.BlockSpec((_N_SUBS, _N_SUBS), lambda i: (0, 0)),
        ],
        out_specs=pl.BlockSpec((B, _D_OUT), lambda i: (i, 0)),
        out_shape=jax.ShapeDtypeStruct((n, _D_OUT), jnp.float32),
        compiler_params=par,
    )(lv, x, sub3, t2)
    return out
